# Initial kernel scaffold; baseline (speedup 1.0000x reference)
import jax
import jax.numpy as jnp
from jax.experimental import pallas as pl


def kernel(x_in, W1, b1, g1, be1, W2, b2, W3, b3, g2, be2, W4, b4):
    raise NotImplementedError("write your pallas kernel here")


# speedup vs baseline: 1.0050x; 1.0050x over previous
import functools

import jax
import jax.numpy as jnp
from jax.experimental import pallas as pl
from jax.experimental.pallas import tpu as pltpu

EPS = 1e-5
K = 16


def _mlp1_body(x_ref, W1_ref, b1_ref, g1_ref, be1_ref, W2_ref, b2_ref, out_ref):
    x = x_ref[...]
    h = jnp.dot(x, W1_ref[...].T, preferred_element_type=jnp.float32) + b1_ref[...][None, :]
    m = jnp.mean(h, axis=0, keepdims=True)
    v = jnp.mean((h - m) ** 2, axis=0, keepdims=True)
    h = g1_ref[...][None, :] * (h - m) / jnp.sqrt(v + EPS) + be1_ref[...][None, :]
    h = jnp.maximum(h, 0.0)
    out_ref[...] = (jnp.dot(h, W2_ref[...].T, preferred_element_type=jnp.float32)
                    + b2_ref[...][None, :])


def _mlp1(x_flat, W1, b1, g1, be1, W2, b2):
    BN = x_flat.shape[0]
    return pl.pallas_call(
        _mlp1_body,
        out_shape=jax.ShapeDtypeStruct((BN, 64), jnp.float32),
    )(x_flat, W1, b1, g1, be1, W2, b2)


def _mlp2_body(x_ref, W3_ref, b3_ref, g2_ref, be2_ref, W4_ref, b4_ref, out_ref):
    x = x_ref[...]
    h = jnp.dot(x, W3_ref[...].T, preferred_element_type=jnp.float32) + b3_ref[...][None, :]
    m = jnp.mean(h, axis=0, keepdims=True)
    v = jnp.mean((h - m) ** 2, axis=0, keepdims=True)
    h = g2_ref[...][None, :] * (h - m) / jnp.sqrt(v + EPS) + be2_ref[...][None, :]
    h = jnp.maximum(h, 0.0)
    out_ref[...] = (jnp.dot(h, W4_ref[...].T, preferred_element_type=jnp.float32)
                    + b4_ref[...][None, :])


def _mlp2(x_flat, W3, b3, g2, be2, W4, b4):
    BS = x_flat.shape[0]
    return pl.pallas_call(
        _mlp2_body,
        out_shape=jax.ShapeDtypeStruct((BS, 64), jnp.float32),
    )(x_flat, W3, b3, g2, be2, W4, b4)


def _index_points(points, idx):
    if idx.ndim == 2:
        return jnp.take_along_axis(points, idx[:, :, None], axis=1)
    B, S, Kn = idx.shape
    flat = idx.reshape(B, S * Kn)
    g = jnp.take_along_axis(points, flat[:, :, None], axis=1)
    return g.reshape(B, S, Kn, points.shape[-1])


def _square_distance(src, dst):
    d = -2.0 * jnp.einsum('bsc,bnc->bsn', src, dst)
    d = d + jnp.sum(src ** 2, axis=-1)[:, :, None]
    d = d + jnp.sum(dst ** 2, axis=-1)[:, None, :]
    return d


def _farthest_point_sample(xyz, npoint):
    B, N, _ = xyz.shape
    def body(i, state):
        distance, farthest, centroids = state
        centroids = centroids.at[:, i].set(farthest)
        centroid = jnp.take_along_axis(xyz, farthest[:, None, None], axis=1)
        dist = jnp.sum((xyz - centroid) ** 2, axis=-1)
        distance = jnp.minimum(distance, dist)
        farthest = jnp.argmax(distance, axis=-1).astype(jnp.int32)
        return (distance, farthest, centroids)
    distance0 = jnp.full((B, N), 1e10, dtype=xyz.dtype)
    farthest0 = jnp.zeros((B,), dtype=jnp.int32)
    centroids0 = jnp.zeros((B, npoint), dtype=jnp.int32)
    _, _, centroids = jax.lax.fori_loop(0, npoint, body, (distance0, farthest0, centroids0))
    return centroids


def kernel(x_in, W1, b1, g1, be1, W2, b2, W3, b3, g2, be2, W4, b4):
    B, N, _ = x_in.shape
    S = N // 2

    feats = _mlp1(x_in.reshape(B * N, 3), W1, b1, g1, be1, W2, b2).reshape(B, N, 64)

    fps_idx = _farthest_point_sample(x_in, S)
    new_xyz = _index_points(x_in, fps_idx)
    d = _square_distance(new_xyz, x_in)
    _, knn_idx = jax.lax.top_k(-d, K)
    grouped_xyz = _index_points(x_in, knn_idx)
    grouped_xyz_norm = grouped_xyz - new_xyz[:, :, None, :]
    grouped_points = _index_points(feats, knn_idx)
    new_points = jnp.concatenate([grouped_xyz_norm, grouped_points], axis=-1)
    x_cat = jnp.max(new_points, axis=-2)

    out = _mlp2(x_cat.reshape(B * S, 67), W3, b3, g2, be2, W4, b4)
    return out.reshape(B, S, 64)


# speedup vs baseline: 2.3207x; 2.3091x over previous
import functools

import jax
import jax.numpy as jnp
from jax.experimental import pallas as pl
from jax.experimental.pallas import tpu as pltpu

EPS = 1e-5
K = 16


def _mlp1_body(x_ref, W1_ref, b1_ref, g1_ref, be1_ref, W2_ref, b2_ref, out_ref):
    x = x_ref[...]
    h = jnp.dot(x, W1_ref[...].T, preferred_element_type=jnp.float32) + b1_ref[...][None, :]
    m = jnp.mean(h, axis=0, keepdims=True)
    v = jnp.mean((h - m) ** 2, axis=0, keepdims=True)
    h = g1_ref[...][None, :] * (h - m) / jnp.sqrt(v + EPS) + be1_ref[...][None, :]
    h = jnp.maximum(h, 0.0)
    out_ref[...] = (jnp.dot(h, W2_ref[...].T, preferred_element_type=jnp.float32)
                    + b2_ref[...][None, :])


def _mlp1(x_flat, W1, b1, g1, be1, W2, b2):
    BN = x_flat.shape[0]
    return pl.pallas_call(
        _mlp1_body,
        out_shape=jax.ShapeDtypeStruct((BN, 64), jnp.float32),
    )(x_flat, W1, b1, g1, be1, W2, b2)


def _mlp2_body(x_ref, W3_ref, b3_ref, g2_ref, be2_ref, W4_ref, b4_ref, out_ref):
    x = x_ref[...]
    h = jnp.dot(x, W3_ref[...].T, preferred_element_type=jnp.float32) + b3_ref[...][None, :]
    m = jnp.mean(h, axis=0, keepdims=True)
    v = jnp.mean((h - m) ** 2, axis=0, keepdims=True)
    h = g2_ref[...][None, :] * (h - m) / jnp.sqrt(v + EPS) + be2_ref[...][None, :]
    h = jnp.maximum(h, 0.0)
    out_ref[...] = (jnp.dot(h, W4_ref[...].T, preferred_element_type=jnp.float32)
                    + b4_ref[...][None, :])


def _mlp2(x_flat, W3, b3, g2, be2, W4, b4):
    BS = x_flat.shape[0]
    return pl.pallas_call(
        _mlp2_body,
        out_shape=jax.ShapeDtypeStruct((BS, 64), jnp.float32),
    )(x_flat, W3, b3, g2, be2, W4, b4)


def _fps_body(x_ref, y_ref, z_ref, cx_ref, cy_ref, cz_ref):
    X = x_ref[...]
    Y = y_ref[...]
    Z = z_ref[...]
    B, N = X.shape
    S = cx_ref.shape[1]
    lane = jax.lax.broadcasted_iota(jnp.int32, (B, N), 1)
    lane128 = jax.lax.broadcasted_iota(jnp.int32, (B, 128), 1)

    def step(j, carry):
        distance, farthest, bufx, bufy, bufz = carry
        sel = lane == farthest
        cx = jnp.sum(jnp.where(sel, X, 0.0), axis=1, keepdims=True)
        cy = jnp.sum(jnp.where(sel, Y, 0.0), axis=1, keepdims=True)
        cz = jnp.sum(jnp.where(sel, Z, 0.0), axis=1, keepdims=True)
        put = lane128 == j
        bufx = jnp.where(put, cx, bufx)
        bufy = jnp.where(put, cy, bufy)
        bufz = jnp.where(put, cz, bufz)
        dx = X - cx
        dy = Y - cy
        dz = Z - cz
        dist = (dx * dx + dy * dy) + dz * dz
        distance = jnp.minimum(distance, dist)
        m = jnp.max(distance, axis=1, keepdims=True)
        nf = jnp.min(jnp.where(distance == m, lane, N), axis=1, keepdims=True)
        return (distance, nf, bufx, bufy, bufz)

    def chunk(c, carry):
        distance, farthest = carry
        buf0 = jnp.zeros((B, 128), jnp.float32)
        distance, farthest, bufx, bufy, bufz = jax.lax.fori_loop(
            0, 128, step, (distance, farthest, buf0, buf0, buf0))
        base = pl.multiple_of(c * 128, 128)
        cx_ref[:, pl.ds(base, 128)] = bufx
        cy_ref[:, pl.ds(base, 128)] = bufy
        cz_ref[:, pl.ds(base, 128)] = bufz
        return (distance, farthest)

    dist0 = jnp.full((B, N), 1e10, jnp.float32)
    f0 = jnp.zeros((B, 1), jnp.int32)
    jax.lax.fori_loop(0, S // 128, chunk, (dist0, f0))


def _fps_pallas(x_in, S, interpret=False):
    B, N, _ = x_in.shape
    xyzT = jnp.transpose(x_in, (2, 0, 1))
    out = pl.pallas_call(
        _fps_body,
        out_shape=[jax.ShapeDtypeStruct((B, S), jnp.float32)] * 3,
        interpret=interpret,
    )(xyzT[0], xyzT[1], xyzT[2])
    return jnp.stack(out, axis=-1)


def _index_points(points, idx):
    if idx.ndim == 2:
        return jnp.take_along_axis(points, idx[:, :, None], axis=1)
    B, S, Kn = idx.shape
    flat = idx.reshape(B, S * Kn)
    g = jnp.take_along_axis(points, flat[:, :, None], axis=1)
    return g.reshape(B, S, Kn, points.shape[-1])


def _square_distance(src, dst):
    d = -2.0 * jnp.einsum('bsc,bnc->bsn', src, dst)
    d = d + jnp.sum(src ** 2, axis=-1)[:, :, None]
    d = d + jnp.sum(dst ** 2, axis=-1)[:, None, :]
    return d


def _farthest_point_sample(xyz, npoint):
    B, N, _ = xyz.shape
    def body(i, state):
        distance, farthest, centroids = state
        centroids = centroids.at[:, i].set(farthest)
        centroid = jnp.take_along_axis(xyz, farthest[:, None, None], axis=1)
        dist = jnp.sum((xyz - centroid) ** 2, axis=-1)
        distance = jnp.minimum(distance, dist)
        farthest = jnp.argmax(distance, axis=-1).astype(jnp.int32)
        return (distance, farthest, centroids)
    distance0 = jnp.full((B, N), 1e10, dtype=xyz.dtype)
    farthest0 = jnp.zeros((B,), dtype=jnp.int32)
    centroids0 = jnp.zeros((B, npoint), dtype=jnp.int32)
    _, _, centroids = jax.lax.fori_loop(0, npoint, body, (distance0, farthest0, centroids0))
    return centroids


def kernel(x_in, W1, b1, g1, be1, W2, b2, W3, b3, g2, be2, W4, b4):
    B, N, _ = x_in.shape
    S = N // 2

    feats = _mlp1(x_in.reshape(B * N, 3), W1, b1, g1, be1, W2, b2).reshape(B, N, 64)

    new_xyz = _fps_pallas(x_in, S)
    d = _square_distance(new_xyz, x_in)
    _, knn_idx = jax.lax.top_k(-d, K)
    grouped_xyz = _index_points(x_in, knn_idx)
    grouped_xyz_norm = grouped_xyz - new_xyz[:, :, None, :]
    grouped_points = _index_points(feats, knn_idx)
    new_points = jnp.concatenate([grouped_xyz_norm, grouped_points], axis=-1)
    x_cat = jnp.max(new_points, axis=-2)

    out = _mlp2(x_cat.reshape(B * S, 67), W3, b3, g2, be2, W4, b4)
    return out.reshape(B, S, 64)


# speedup vs baseline: 4.7941x; 2.0658x over previous
import functools

import jax
import jax.numpy as jnp
from jax.experimental import pallas as pl
from jax.experimental.pallas import tpu as pltpu

EPS = 1e-5
K = 16


def _mlp1_body(x_ref, W1_ref, b1_ref, g1_ref, be1_ref, W2_ref, b2_ref, out_ref):
    x = x_ref[...]
    h = jnp.dot(x, W1_ref[...].T, preferred_element_type=jnp.float32) + b1_ref[...][None, :]
    m = jnp.mean(h, axis=0, keepdims=True)
    v = jnp.mean((h - m) ** 2, axis=0, keepdims=True)
    h = g1_ref[...][None, :] * (h - m) / jnp.sqrt(v + EPS) + be1_ref[...][None, :]
    h = jnp.maximum(h, 0.0)
    out_ref[...] = (jnp.dot(h, W2_ref[...].T, preferred_element_type=jnp.float32)
                    + b2_ref[...][None, :])


def _mlp1(x_flat, W1, b1, g1, be1, W2, b2):
    BN = x_flat.shape[0]
    return pl.pallas_call(
        _mlp1_body,
        out_shape=jax.ShapeDtypeStruct((BN, 64), jnp.float32),
    )(x_flat, W1, b1, g1, be1, W2, b2)


def _mlp2_body(x_ref, W3_ref, b3_ref, g2_ref, be2_ref, W4_ref, b4_ref, out_ref):
    x = x_ref[...]
    h = jnp.dot(x, W3_ref[...].T, preferred_element_type=jnp.float32) + b3_ref[...][None, :]
    m = jnp.mean(h, axis=0, keepdims=True)
    v = jnp.mean((h - m) ** 2, axis=0, keepdims=True)
    h = g2_ref[...][None, :] * (h - m) / jnp.sqrt(v + EPS) + be2_ref[...][None, :]
    h = jnp.maximum(h, 0.0)
    out_ref[...] = (jnp.dot(h, W4_ref[...].T, preferred_element_type=jnp.float32)
                    + b4_ref[...][None, :])


def _mlp2(x_flat, W3, b3, g2, be2, W4, b4):
    BS = x_flat.shape[0]
    return pl.pallas_call(
        _mlp2_body,
        out_shape=jax.ShapeDtypeStruct((BS, 64), jnp.float32),
    )(x_flat, W3, b3, g2, be2, W4, b4)


def _fps_body(x_ref, y_ref, z_ref, cx_ref, cy_ref, cz_ref):
    X = x_ref[...]
    Y = y_ref[...]
    Z = z_ref[...]
    B, N = X.shape
    S = cx_ref.shape[1]
    lane = jax.lax.broadcasted_iota(jnp.int32, (B, N), 1)
    lane128 = jax.lax.broadcasted_iota(jnp.int32, (B, 128), 1)

    def step(j, carry):
        distance, farthest, bufx, bufy, bufz = carry
        sel = lane == farthest
        cx = jnp.sum(jnp.where(sel, X, 0.0), axis=1, keepdims=True)
        cy = jnp.sum(jnp.where(sel, Y, 0.0), axis=1, keepdims=True)
        cz = jnp.sum(jnp.where(sel, Z, 0.0), axis=1, keepdims=True)
        put = lane128 == j
        bufx = jnp.where(put, cx, bufx)
        bufy = jnp.where(put, cy, bufy)
        bufz = jnp.where(put, cz, bufz)
        dx = X - cx
        dy = Y - cy
        dz = Z - cz
        dist = (dx * dx + dy * dy) + dz * dz
        distance = jnp.minimum(distance, dist)
        m = jnp.max(distance, axis=1, keepdims=True)
        nf = jnp.min(jnp.where(distance == m, lane, N), axis=1, keepdims=True)
        return (distance, nf, bufx, bufy, bufz)

    def chunk(c, carry):
        distance, farthest = carry
        buf0 = jnp.zeros((B, 128), jnp.float32)
        distance, farthest, bufx, bufy, bufz = jax.lax.fori_loop(
            0, 128, step, (distance, farthest, buf0, buf0, buf0))
        base = pl.multiple_of(c * 128, 128)
        cx_ref[:, pl.ds(base, 128)] = bufx
        cy_ref[:, pl.ds(base, 128)] = bufy
        cz_ref[:, pl.ds(base, 128)] = bufz
        return (distance, farthest)

    dist0 = jnp.full((B, N), 1e10, jnp.float32)
    f0 = jnp.zeros((B, 1), jnp.int32)
    jax.lax.fori_loop(0, S // 128, chunk, (dist0, f0))


def _fps_pallas(x_in, S, interpret=False):
    B, N, _ = x_in.shape
    xyzT = jnp.transpose(x_in, (2, 0, 1))
    out = pl.pallas_call(
        _fps_body,
        out_shape=[jax.ShapeDtypeStruct((B, S), jnp.float32)] * 3,
        interpret=interpret,
    )(xyzT[0], xyzT[1], xyzT[2])
    return jnp.stack(out, axis=-1)


_SBLK = 256


def _knn_body(cxyz_ref, xyzt_ref, idx_ref):
    C = cxyz_ref[0]
    Xt = xyzt_ref[0]
    N = Xt.shape[1]
    b = pl.program_id(0)
    dot = jnp.dot(C, Xt, preferred_element_type=jnp.float32)
    cc = jnp.sum(C * C, axis=1, keepdims=True)
    xx = jnp.sum(Xt * Xt, axis=0, keepdims=True)
    d = (-2.0 * dot + cc) + xx
    bits = jax.lax.bitcast_convert_type(d, jnp.int32)
    keys = bits ^ ((bits >> 31) & jnp.int32(0x7FFFFFFF))
    lane = jax.lax.broadcasted_iota(jnp.int32, keys.shape, 1)
    imax = jnp.int32(0x7FFFFFFF)
    for t in range(K):
        m = jnp.min(keys, axis=1, keepdims=True)
        idx = jnp.min(jnp.where(keys == m, lane, N), axis=1, keepdims=True)
        keys = jnp.where(lane == idx, imax, keys)
        idx_ref[0, :, pl.ds(t, 1)] = idx + b * N


def _knn_pallas(new_xyz, x_in, interpret=False):
    B, S, _ = new_xyz.shape
    N = x_in.shape[1]
    xyzt = jnp.transpose(x_in, (0, 2, 1))
    grid = (B, S // _SBLK)
    return pl.pallas_call(
        _knn_body,
        grid=grid,
        in_specs=[
            pl.BlockSpec((1, _SBLK, 3), lambda b, s: (b, s, 0)),
            pl.BlockSpec((1, 3, N), lambda b, s: (b, 0, 0)),
        ],
        out_specs=pl.BlockSpec((1, _SBLK, K), lambda b, s: (b, s, 0)),
        out_shape=jax.ShapeDtypeStruct((B, S, K), jnp.int32),
        interpret=interpret,
    )(new_xyz, xyzt)


def _index_points(points, idx):
    if idx.ndim == 2:
        return jnp.take_along_axis(points, idx[:, :, None], axis=1)
    B, S, Kn = idx.shape
    flat = idx.reshape(B, S * Kn)
    g = jnp.take_along_axis(points, flat[:, :, None], axis=1)
    return g.reshape(B, S, Kn, points.shape[-1])


def _square_distance(src, dst):
    d = -2.0 * jnp.einsum('bsc,bnc->bsn', src, dst)
    d = d + jnp.sum(src ** 2, axis=-1)[:, :, None]
    d = d + jnp.sum(dst ** 2, axis=-1)[:, None, :]
    return d


def _farthest_point_sample(xyz, npoint):
    B, N, _ = xyz.shape
    def body(i, state):
        distance, farthest, centroids = state
        centroids = centroids.at[:, i].set(farthest)
        centroid = jnp.take_along_axis(xyz, farthest[:, None, None], axis=1)
        dist = jnp.sum((xyz - centroid) ** 2, axis=-1)
        distance = jnp.minimum(distance, dist)
        farthest = jnp.argmax(distance, axis=-1).astype(jnp.int32)
        return (distance, farthest, centroids)
    distance0 = jnp.full((B, N), 1e10, dtype=xyz.dtype)
    farthest0 = jnp.zeros((B,), dtype=jnp.int32)
    centroids0 = jnp.zeros((B, npoint), dtype=jnp.int32)
    _, _, centroids = jax.lax.fori_loop(0, npoint, body, (distance0, farthest0, centroids0))
    return centroids


def kernel(x_in, W1, b1, g1, be1, W2, b2, W3, b3, g2, be2, W4, b4):
    B, N, _ = x_in.shape
    S = N // 2

    feats = _mlp1(x_in.reshape(B * N, 3), W1, b1, g1, be1, W2, b2).reshape(B, N, 64)

    new_xyz = _fps_pallas(x_in, S)
    knn_glob = _knn_pallas(new_xyz, x_in)
    knn_idx = knn_glob - (jnp.arange(B, dtype=jnp.int32)[:, None, None]) * N
    grouped_xyz = _index_points(x_in, knn_idx)
    grouped_xyz_norm = grouped_xyz - new_xyz[:, :, None, :]
    grouped_points = _index_points(feats, knn_idx)
    new_points = jnp.concatenate([grouped_xyz_norm, grouped_points], axis=-1)
    x_cat = jnp.max(new_points, axis=-2)

    out = _mlp2(x_cat.reshape(B * S, 67), W3, b3, g2, be2, W4, b4)
    return out.reshape(B, S, 64)


# speedup vs baseline: 17.3826x; 3.6259x over previous
import functools

import jax
import jax.numpy as jnp
from jax import lax
from jax.experimental import pallas as pl
from jax.experimental.pallas import tpu as pltpu
from jax.experimental.pallas import tpu_sc as plsc

EPS = 1e-5
K = 16
TW = 128


def _mlp1_body(x_ref, W1_ref, b1_ref, g1_ref, be1_ref, W2_ref, b2_ref, out_ref):
    x = x_ref[...]
    h = jnp.dot(x, W1_ref[...].T, preferred_element_type=jnp.float32) + b1_ref[...][None, :]
    m = jnp.mean(h, axis=0, keepdims=True)
    v = jnp.mean((h - m) ** 2, axis=0, keepdims=True)
    h = g1_ref[...][None, :] * (h - m) / jnp.sqrt(v + EPS) + be1_ref[...][None, :]
    h = jnp.maximum(h, 0.0)
    feats = (jnp.dot(h, W2_ref[...].T, preferred_element_type=jnp.float32)
             + b2_ref[...][None, :])
    pad = jnp.zeros((x.shape[0], TW - 67), jnp.float32)
    out_ref[...] = jnp.concatenate([feats, x, pad], axis=1)


def _mlp1(x_flat, W1, b1, g1, be1, W2, b2):
    BN = x_flat.shape[0]
    return pl.pallas_call(
        _mlp1_body,
        out_shape=jax.ShapeDtypeStruct((BN, TW), jnp.float32),
    )(x_flat, W1, b1, g1, be1, W2, b2)


def _mlp2_body(mg_ref, nx_ref, Wp_ref, W3xT_ref, b3_ref, g2_ref, be2_ref,
               W4_ref, b4_ref, out_ref):
    mg = mg_ref[...]
    nx = nx_ref[...]
    h = (jnp.dot(mg, Wp_ref[...], preferred_element_type=jnp.float32)
         - jnp.dot(nx, W3xT_ref[...], preferred_element_type=jnp.float32)
         + b3_ref[...][None, :])
    m = jnp.mean(h, axis=0, keepdims=True)
    v = jnp.mean((h - m) ** 2, axis=0, keepdims=True)
    h = g2_ref[...][None, :] * (h - m) / jnp.sqrt(v + EPS) + be2_ref[...][None, :]
    h = jnp.maximum(h, 0.0)
    out_ref[...] = (jnp.dot(h, W4_ref[...].T, preferred_element_type=jnp.float32)
                    + b4_ref[...][None, :])


def _mlp2(mg, nx, W3, b3, g2, be2, W4, b4):
    BS = mg.shape[0]
    Wp = jnp.zeros((TW, 64), jnp.float32)
    Wp = Wp.at[0:64].set(W3[:, 3:].T).at[64:67].set(W3[:, :3].T)
    W3xT = W3[:, :3].T
    return pl.pallas_call(
        _mlp2_body,
        out_shape=jax.ShapeDtypeStruct((BS, 64), jnp.float32),
    )(mg, nx, Wp, W3xT, b3, g2, be2, W4, b4)


_SC_CHUNK = 8


def _sc_gather_max(table, idx_flat, BS):
    info = plsc.get_sparse_core_info()
    NC, NS = info.num_cores, info.num_subcores
    NW = NC * NS
    per_w = BS // NW
    n_chunks = per_w // _SC_CHUNK
    mesh = plsc.VectorSubcoreMesh(core_axis_name="c", subcore_axis_name="s")

    @functools.partial(
        pl.kernel, mesh=mesh,
        out_type=jax.ShapeDtypeStruct((BS, TW), jnp.float32),
        scratch_types=[
            pltpu.VMEM((_SC_CHUNK * K,), jnp.int32),
            pltpu.VMEM((_SC_CHUNK * K, TW), jnp.float32),
            pltpu.VMEM((_SC_CHUNK, TW), jnp.float32),
            pltpu.SemaphoreType.DMA,
        ],
    )
    def k(table_hbm, idx_hbm, out_hbm, idx_v, rows_v, out_v, sem):
        wid = lax.axis_index("s") * NC + lax.axis_index("c")
        base = wid * per_w

        def chunk_body(chunk, _):
            cbase = base + chunk * _SC_CHUNK
            pltpu.sync_copy(idx_hbm.at[pl.ds(cbase * K, _SC_CHUNK * K)], idx_v)
            pltpu.async_copy(table_hbm.at[idx_v], rows_v, sem).wait()

            def ci_body(ci, _):
                for g in range(TW // 16):
                    acc = rows_v[ci * K, pl.ds(g * 16, 16)]
                    for r in range(1, K):
                        acc = jnp.maximum(acc, rows_v[ci * K + r, pl.ds(g * 16, 16)])
                    out_v[ci, pl.ds(g * 16, 16)] = acc
                return 0

            lax.fori_loop(0, _SC_CHUNK, ci_body, 0)
            pltpu.sync_copy(out_v, out_hbm.at[pl.ds(cbase, _SC_CHUNK)])
            return 0

        lax.fori_loop(0, n_chunks, chunk_body, 0)

    return k(table, idx_flat)


def _fps_body(x_ref, y_ref, z_ref, cx_ref, cy_ref, cz_ref):
    X = x_ref[...]
    Y = y_ref[...]
    Z = z_ref[...]
    B, N = X.shape
    S = cx_ref.shape[1]
    lane = jax.lax.broadcasted_iota(jnp.int32, (B, N), 1)
    lane128 = jax.lax.broadcasted_iota(jnp.int32, (B, 128), 1)

    def step(j, carry):
        distance, farthest, bufx, bufy, bufz = carry
        sel = lane == farthest
        cx = jnp.sum(jnp.where(sel, X, 0.0), axis=1, keepdims=True)
        cy = jnp.sum(jnp.where(sel, Y, 0.0), axis=1, keepdims=True)
        cz = jnp.sum(jnp.where(sel, Z, 0.0), axis=1, keepdims=True)
        put = lane128 == j
        bufx = jnp.where(put, cx, bufx)
        bufy = jnp.where(put, cy, bufy)
        bufz = jnp.where(put, cz, bufz)
        dx = X - cx
        dy = Y - cy
        dz = Z - cz
        dist = dx * dx + (dy * dy + dz * dz)
        distance = jnp.minimum(distance, dist)
        m = jnp.max(distance, axis=1, keepdims=True)
        nf = jnp.min(jnp.where(distance == m, lane, N), axis=1, keepdims=True)
        return (distance, nf, bufx, bufy, bufz)

    def chunk(c, carry):
        distance, farthest = carry
        buf0 = jnp.zeros((B, 128), jnp.float32)
        distance, farthest, bufx, bufy, bufz = jax.lax.fori_loop(
            0, 128, step, (distance, farthest, buf0, buf0, buf0))
        base = pl.multiple_of(c * 128, 128)
        cx_ref[:, pl.ds(base, 128)] = bufx
        cy_ref[:, pl.ds(base, 128)] = bufy
        cz_ref[:, pl.ds(base, 128)] = bufz
        return (distance, farthest)

    dist0 = jnp.full((B, N), 1e10, jnp.float32)
    f0 = jnp.zeros((B, 1), jnp.int32)
    jax.lax.fori_loop(0, S // 128, chunk, (dist0, f0))


def _fps_pallas(x_in, S, interpret=False):
    B, N, _ = x_in.shape
    xyzT = jnp.transpose(x_in, (2, 0, 1))
    out = pl.pallas_call(
        _fps_body,
        out_shape=[jax.ShapeDtypeStruct((B, S), jnp.float32)] * 3,
        interpret=interpret,
    )(xyzT[0], xyzT[1], xyzT[2])
    return jnp.stack(out, axis=-1)


_SBLK = 256


def _knn_body(cxyz_ref, xyzt_ref, idx_ref):
    C = cxyz_ref[0]
    Xt = xyzt_ref[0]
    N = Xt.shape[1]
    b = pl.program_id(0)
    dot = jnp.dot(C, Xt, preferred_element_type=jnp.float32)
    cc = jnp.sum(C * C, axis=1, keepdims=True)
    xx = jnp.sum(Xt * Xt, axis=0, keepdims=True)
    d = (-2.0 * dot + cc) + xx
    bits = jax.lax.bitcast_convert_type(d, jnp.int32)
    keys = bits ^ ((bits >> 31) & jnp.int32(0x7FFFFFFF))
    lane = jax.lax.broadcasted_iota(jnp.int32, keys.shape, 1)
    imax = jnp.int32(0x7FFFFFFF)
    for t in range(K):
        m = jnp.min(keys, axis=1, keepdims=True)
        idx = jnp.min(jnp.where(keys == m, lane, N), axis=1, keepdims=True)
        keys = jnp.where(lane == idx, imax, keys)
        idx_ref[0, :, pl.ds(t, 1)] = idx + b * N


def _knn_pallas(new_xyz, x_in, interpret=False):
    B, S, _ = new_xyz.shape
    N = x_in.shape[1]
    xyzt = jnp.transpose(x_in, (0, 2, 1))
    grid = (B, S // _SBLK)
    return pl.pallas_call(
        _knn_body,
        grid=grid,
        in_specs=[
            pl.BlockSpec((1, _SBLK, 3), lambda b, s: (b, s, 0)),
            pl.BlockSpec((1, 3, N), lambda b, s: (b, 0, 0)),
        ],
        out_specs=pl.BlockSpec((1, _SBLK, K), lambda b, s: (b, s, 0)),
        out_shape=jax.ShapeDtypeStruct((B, S, K), jnp.int32),
        interpret=interpret,
    )(new_xyz, xyzt)


def _index_points(points, idx):
    if idx.ndim == 2:
        return jnp.take_along_axis(points, idx[:, :, None], axis=1)
    B, S, Kn = idx.shape
    flat = idx.reshape(B, S * Kn)
    g = jnp.take_along_axis(points, flat[:, :, None], axis=1)
    return g.reshape(B, S, Kn, points.shape[-1])


def _square_distance(src, dst):
    d = -2.0 * jnp.einsum('bsc,bnc->bsn', src, dst)
    d = d + jnp.sum(src ** 2, axis=-1)[:, :, None]
    d = d + jnp.sum(dst ** 2, axis=-1)[:, None, :]
    return d


def _farthest_point_sample(xyz, npoint):
    B, N, _ = xyz.shape
    def body(i, state):
        distance, farthest, centroids = state
        centroids = centroids.at[:, i].set(farthest)
        centroid = jnp.take_along_axis(xyz, farthest[:, None, None], axis=1)
        dist = jnp.sum((xyz - centroid) ** 2, axis=-1)
        distance = jnp.minimum(distance, dist)
        farthest = jnp.argmax(distance, axis=-1).astype(jnp.int32)
        return (distance, farthest, centroids)
    distance0 = jnp.full((B, N), 1e10, dtype=xyz.dtype)
    farthest0 = jnp.zeros((B,), dtype=jnp.int32)
    centroids0 = jnp.zeros((B, npoint), dtype=jnp.int32)
    _, _, centroids = jax.lax.fori_loop(0, npoint, body, (distance0, farthest0, centroids0))
    return centroids


def kernel(x_in, W1, b1, g1, be1, W2, b2, W3, b3, g2, be2, W4, b4):
    B, N, _ = x_in.shape
    S = N // 2

    table = _mlp1(x_in.reshape(B * N, 3), W1, b1, g1, be1, W2, b2)
    new_xyz = _fps_pallas(x_in, S)
    knn_glob = _knn_pallas(new_xyz, x_in)
    mg = _sc_gather_max(table, knn_glob.reshape(B * S * K), B * S)
    out = _mlp2(mg, new_xyz.reshape(B * S, 3), W3, b3, g2, be2, W4, b4)
    return out.reshape(B, S, 64)
